# baseline (speedup 1.0000x reference)
import jax
import jax.numpy as jnp
from jax import lax
from jax.experimental import pallas as pl
from jax.experimental.pallas import tpu as pltpu
from jax.experimental.pallas import tpu_sc as plsc

VOCAB = 50265
HID = 768
PAD = 1
L = 16
NVR = HID // L
NC, NS = 2, 16
NW = NC * NS
CH = 32
OS = 4

_RDOWN = [(2.0**64, 2.0**-64, 2.0**-32), (2.0**32, 2.0**-32, 2.0**-16),
          (2.0**16, 2.0**-16, 2.0**-8), (2.0**8, 2.0**-8, 2.0**-4),
          (2.0**4, 2.0**-4, 2.0**-2), (2.0**2, 2.0**-2, 2.0**-1)]
_RUP = [(2.0**-16, 2.0**16, 2.0**8), (2.0**-8, 2.0**8, 2.0**4),
        (2.0**-4, 2.0**4, 2.0**2), (2.0**-2, 2.0**2, 2.0**1)]


def _perm(v, idx):
    return v.at[idx].get(mode="promise_in_bounds")


def _allsum(v):
    iota = lax.iota(jnp.int32, L)
    for k in (8, 4, 2, 1):
        v = v + _perm(v, jnp.bitwise_xor(iota, k))
    return v


def _prefix_sum(c):
    iota = lax.iota(jnp.int32, L)
    for k in (1, 2, 4, 8):
        zm = jnp.minimum(jnp.maximum(iota - (k - 1), 0), 1)
        c = c + _perm(c, jnp.maximum(iota - k, 0)) * zm
    return c


def _rsqrt_vec(a):
    m = a
    r = jnp.full((L,), 1.0, jnp.float32)
    for th, mm, rm in _RDOWN:
        ind = jnp.where(m >= th, 1.0, 0.0)
        m = m * (1.0 + ind * (mm - 1.0))
        r = r * (1.0 + ind * (rm - 1.0))
    for th, mm, rm in _RUP:
        ind = jnp.where(m < th, 1.0, 0.0)
        m = m * (1.0 + ind * (mm - 1.0))
        r = r * (1.0 + ind * (rm - 1.0))
    y = jnp.full((L,), 0.7, jnp.float32)
    for _ in range(6):
        y = y * (1.5 - 0.5 * m * y * y)
    return y * r


def _make_pooler(R, T):
    rpw = R // NW
    ptrows = -(-(T + 2) // 8) * 8
    nid = T // L

    def body(ids_hbm, word_hbm, pos_hbm, type_hbm, gam_hbm, bet_hbm, out_hbm,
             pt_v, ty_v, gam_v, bet_v, ids_v, idx_v, g_v, pos_v, eb_v,
             stage_v, sem0, sem1):
        wid = lax.axis_index("s") * NC + lax.axis_index("c")
        row0 = wid * rpw

        pltpu.sync_copy(pos_hbm.at[pl.ds(0, ptrows)], pt_v)
        pltpu.sync_copy(type_hbm, ty_v)
        pltpu.sync_copy(gam_hbm, gam_v)
        pltpu.sync_copy(bet_hbm, bet_v)

        def add_type(rr, _):
            for j in range(NVR):
                sl = pl.ds(j * L, L)
                pt_v[rr, sl] = pt_v[rr, sl] + ty_v[0, sl]
            return 0
        lax.fori_loop(0, ptrows, add_type, 0)

        def load_chunk(ci):
            pltpu.sync_copy(ids_hbm.at[pl.ds(row0 + ci * CH, CH)], ids_v)

        def issue(slot, row, sem):
            rm = lax.rem(row, CH)
            for k in range(nid):
                v = ids_v[rm, pl.ds(k * L, L)]
                v = jnp.minimum(jnp.maximum(v, 0), VOCAB - 1)
                idx_v[slot, pl.ds(k * L, L)] = v
            pltpu.async_copy(word_hbm.at[idx_v.at[slot]], g_v.at[slot], sem)

        def wait(slot, sem):
            pltpu.make_async_copy(
                word_hbm.at[idx_v.at[slot]], g_v.at[slot], sem).wait()

        def process(slot, row):
            rm = lax.rem(row, CH)
            zf = jnp.zeros((L,), jnp.float32)

            cnt = jnp.zeros((L,), jnp.int32)
            carry = jnp.zeros((L,), jnp.int32)
            mf = []
            for k in range(nid):
                v = ids_v[rm, pl.ds(k * L, L)]
                v = jnp.minimum(jnp.maximum(v, 0), VOCAB - 1)
                m = jnp.minimum(jnp.abs(v - PAD), 1)
                c = _prefix_sum(m) + carry
                pos_v[0, pl.ds(k * L, L)] = c * m + PAD
                mf.append(m.astype(jnp.float32))
                tot = _allsum(m)
                carry = carry + tot
                cnt = cnt + tot
            invc_v = 1.0 / cnt.astype(jnp.float32)

            lane0 = jnp.zeros((L,), jnp.int32)
            iota = lax.iota(jnp.int32, L)

            def _sel_group(t, vecs):
                r = vecs[0]
                for k in range(1, nid):
                    gi = jnp.where(t // L == k, 1.0, 0.0)
                    r = r + (vecs[k] - r) * gi
                return r

            def _pos_at(t):
                return pos_v[0, pl.ds(t, L)][0]

            def t1body(t, sq):
                pos_t = _pos_at(t)
                s = zf
                q = zf
                for j in range(NVR):
                    sl = pl.ds(j * L, L)
                    e = g_v[slot, t, sl] + pt_v[pos_t, sl]
                    eb_v[t, sl] = e
                    s = s + e
                    q = q + e * e
                lane_ind = jnp.where(iota == lax.rem(t, L), 1.0, 0.0)
                new = list(sq)
                sall = _allsum(s)
                qall = _allsum(q)
                for k in range(nid):
                    gi = jnp.where(t // L == k, 1.0, 0.0)
                    ind = lane_ind * gi
                    new[k] = sq[k] + (sall - sq[k]) * ind
                    new[nid + k] = sq[nid + k] + (qall - sq[nid + k]) * ind
                return tuple(new)
            sq = lax.fori_loop(0, T, t1body, (zf,) * (2 * nid))

            corr = zf
            cbl = []
            for k in range(nid):
                muv = sq[k] * (1.0 / HID)
                varv = sq[nid + k] * (1.0 / HID) - muv * muv
                cvec = _rsqrt_vec(varv + 1e-5) * mf[k]
                cbl.append(cvec)
                corr = corr + _allsum(cvec * muv)

            def t2body(t, accs):
                ct = _perm(_sel_group(t, cbl), lax.rem(t, L) + lane0)
                new = []
                for j in range(NVR):
                    sl = pl.ds(j * L, L)
                    new.append(accs[j] + ct * eb_v[t, sl])
                return tuple(new)
            accs = lax.fori_loop(0, T, t2body, (zf,) * NVR)

            ro = lax.rem(row, OS)
            for j in range(NVR):
                sl = pl.ds(j * L, L)
                stage_v[ro, sl] = ((accs[j] - corr) * invc_v * gam_v[sl]
                                   + bet_v[sl])

            @pl.when(ro == OS - 1)
            def _():
                base = pl.multiple_of(row0 + row - (OS - 1), OS)
                pltpu.sync_copy(stage_v, out_hbm.at[pl.ds(base, OS)])

        load_chunk(0)
        issue(0, 0, sem0)
        issue(1, 1, sem1)

        def rbody(r2, _):
            ra = 2 * r2
            rb = ra + 1
            wait(0, sem0)
            process(0, ra)

            @pl.when(jnp.logical_and(ra + 2 < rpw, lax.rem(ra + 2, CH) == 0))
            def _():
                load_chunk((ra + 2) // CH)

            @pl.when(ra + 2 < rpw)
            def _():
                issue(0, ra + 2, sem0)

            wait(1, sem1)
            process(1, rb)

            @pl.when(jnp.logical_and(rb + 2 < rpw, lax.rem(rb + 2, CH) == 0))
            def _():
                load_chunk((rb + 2) // CH)

            @pl.when(rb + 2 < rpw)
            def _():
                issue(1, rb + 2, sem1)
            return 0

        lax.fori_loop(0, rpw // 2, rbody, 0)

    mesh = plsc.VectorSubcoreMesh(core_axis_name="c", subcore_axis_name="s")
    return pl.kernel(
        body,
        out_type=jax.ShapeDtypeStruct((R, HID), jnp.float32),
        mesh=mesh,
        scratch_types=[
            pltpu.VMEM((ptrows, HID), jnp.float32),
            pltpu.VMEM((1, HID), jnp.float32),
            pltpu.VMEM((HID,), jnp.float32),
            pltpu.VMEM((HID,), jnp.float32),
            pltpu.VMEM((CH, T), jnp.int32),
            pltpu.VMEM((2, T), jnp.int32),
            pltpu.VMEM((2, T, HID), jnp.float32),
            pltpu.VMEM((1, 48), jnp.int32),
            pltpu.VMEM((T, HID), jnp.float32),
            pltpu.VMEM((OS, HID), jnp.float32),
            pltpu.SemaphoreType.DMA,
            pltpu.SemaphoreType.DMA,
        ],
        name=f"embed_pool_T{T}",
    )


def kernel(x, edge_attr, word_emb, pos_emb, type_emb, ln_gamma, ln_beta):
    ids_x = x[:, 128:144].astype(jnp.int32)
    ids_e = edge_attr[:, 32:64].astype(jnp.int32)
    pooled_x = _make_pooler(x.shape[0], 16)(
        ids_x, word_emb, pos_emb, type_emb, ln_gamma, ln_beta)
    pooled_e = _make_pooler(edge_attr.shape[0], 32)(
        ids_e, word_emb, pos_emb, type_emb, ln_gamma, ln_beta)
    return (jnp.concatenate([x[:, :128], pooled_x], axis=1),
            jnp.concatenate([edge_attr[:, :32], pooled_e], axis=1))


# speedup vs baseline: 1.3530x; 1.3530x over previous
import jax
import jax.numpy as jnp
from jax import lax
from jax.experimental import pallas as pl
from jax.experimental.pallas import tpu as pltpu
from jax.experimental.pallas import tpu_sc as plsc

VOCAB = 50265
HID = 768
PAD = 1
L = 16
NVR = HID // L
NC, NS = 2, 16
NW = NC * NS
CH = 128
OS = 8

_RDOWN = [(2.0**64, 2.0**-64, 2.0**-32), (2.0**32, 2.0**-32, 2.0**-16),
          (2.0**16, 2.0**-16, 2.0**-8), (2.0**8, 2.0**-8, 2.0**-4),
          (2.0**4, 2.0**-4, 2.0**-2), (2.0**2, 2.0**-2, 2.0**-1)]
_RUP = [(2.0**-16, 2.0**16, 2.0**8), (2.0**-8, 2.0**8, 2.0**4),
        (2.0**-4, 2.0**4, 2.0**2), (2.0**-2, 2.0**2, 2.0**1)]


def _perm(v, idx):
    return v.at[idx].get(mode="promise_in_bounds")


def _allsum(v):
    iota = lax.iota(jnp.int32, L)
    for k in (8, 4, 2, 1):
        v = v + _perm(v, jnp.bitwise_xor(iota, k))
    return v


def _prefix_sum(c):
    iota = lax.iota(jnp.int32, L)
    for k in (1, 2, 4, 8):
        zm = jnp.minimum(jnp.maximum(iota - (k - 1), 0), 1)
        c = c + _perm(c, jnp.maximum(iota - k, 0)) * zm
    return c


def _rsqrt_vec(a):
    m = a
    r = jnp.full((L,), 1.0, jnp.float32)
    for th, mm, rm in _RDOWN:
        ind = jnp.where(m >= th, 1.0, 0.0)
        m = m * (1.0 + ind * (mm - 1.0))
        r = r * (1.0 + ind * (rm - 1.0))
    for th, mm, rm in _RUP:
        ind = jnp.where(m < th, 1.0, 0.0)
        m = m * (1.0 + ind * (mm - 1.0))
        r = r * (1.0 + ind * (rm - 1.0))
    y = jnp.full((L,), 0.7, jnp.float32)
    for _ in range(6):
        y = y * (1.5 - 0.5 * m * y * y)
    return y * r


def _make_pooler(R, T):
    rpw = R // NW
    ptrows = -(-(T + 2) // 8) * 8
    nid = T // L

    def body(ids_hbm, word_hbm, pos_hbm, type_hbm, gam_hbm, bet_hbm, out_hbm,
             pt_v, ty_v, gam_v, bet_v, ids_v, idx_v, g_v, pos_v,
             stage_v, sem0, sem1):
        wid = lax.axis_index("s") * NC + lax.axis_index("c")
        row0 = wid * rpw

        pltpu.sync_copy(pos_hbm.at[pl.ds(0, ptrows)], pt_v)
        pltpu.sync_copy(type_hbm, ty_v)
        pltpu.sync_copy(gam_hbm, gam_v)
        pltpu.sync_copy(bet_hbm, bet_v)

        def add_type(rr, _):
            for j in range(NVR):
                sl = pl.ds(j * L, L)
                pt_v[rr, sl] = pt_v[rr, sl] + ty_v[0, sl]
            return 0
        lax.fori_loop(0, ptrows, add_type, 0)

        def load_chunk(ci):
            pltpu.sync_copy(ids_hbm.at[pl.ds(row0 + ci * CH, CH)], ids_v)

        def issue(slot, row, sem):
            rm = lax.rem(row, CH)
            for k in range(nid):
                v = ids_v[rm, pl.ds(k * L, L)]
                v = jnp.minimum(jnp.maximum(v, 0), VOCAB - 1)
                idx_v[slot, pl.ds(k * L, L)] = v
            pltpu.async_copy(word_hbm.at[idx_v.at[slot]], g_v.at[slot], sem)

        def wait(slot, sem):
            pltpu.make_async_copy(
                word_hbm.at[idx_v.at[slot]], g_v.at[slot], sem).wait()

        def process(slot, row):
            rm = lax.rem(row, CH)
            zf = jnp.zeros((L,), jnp.float32)

            cnt = jnp.zeros((L,), jnp.int32)
            carry = jnp.zeros((L,), jnp.int32)
            mf = []
            for k in range(nid):
                v = ids_v[rm, pl.ds(k * L, L)]
                v = jnp.minimum(jnp.maximum(v, 0), VOCAB - 1)
                m = jnp.minimum(jnp.abs(v - PAD), 1)
                c = _prefix_sum(m) + carry
                pos_v[0, pl.ds(k * L, L)] = c * m + PAD
                mf.append(m.astype(jnp.float32))
                tot = _allsum(m)
                carry = carry + tot
                cnt = cnt + tot
            invc_v = 1.0 / cnt.astype(jnp.float32)

            lane0 = jnp.zeros((L,), jnp.int32)
            iota = lax.iota(jnp.int32, L)

            def _sel_group(t, vecs):
                r = vecs[0]
                for k in range(1, nid):
                    gi = jnp.where(t // L == k, 1.0, 0.0)
                    r = r + (vecs[k] - r) * gi
                return r

            def _pos_at(t):
                return pos_v[0, pl.ds(t, L)][0]

            def t1body(t, sq):
                pos_t = _pos_at(t)
                s = zf
                q = zf
                for j in range(NVR):
                    sl = pl.ds(j * L, L)
                    e = g_v[slot, t, sl] + pt_v[pos_t, sl]
                    s = s + e
                    q = q + e * e
                lane_ind = jnp.where(iota == lax.rem(t, L), 1.0, 0.0)
                new = list(sq)
                sall = _allsum(s)
                qall = _allsum(q)
                for k in range(nid):
                    gi = jnp.where(t // L == k, 1.0, 0.0)
                    ind = lane_ind * gi
                    new[k] = sq[k] + (sall - sq[k]) * ind
                    new[nid + k] = sq[nid + k] + (qall - sq[nid + k]) * ind
                return tuple(new)
            sq = lax.fori_loop(0, T, t1body, (zf,) * (2 * nid))

            corr = zf
            cbl = []
            for k in range(nid):
                muv = sq[k] * (1.0 / HID)
                varv = sq[nid + k] * (1.0 / HID) - muv * muv
                cvec = _rsqrt_vec(varv + 1e-5) * mf[k]
                cbl.append(cvec)
                corr = corr + _allsum(cvec * muv)

            def t2body(t, accs):
                ct = _perm(_sel_group(t, cbl), lax.rem(t, L) + lane0)
                pos_t = _pos_at(t)
                new = []
                for j in range(NVR):
                    sl = pl.ds(j * L, L)
                    e = g_v[slot, t, sl] + pt_v[pos_t, sl]
                    new.append(accs[j] + ct * e)
                return tuple(new)
            accs = lax.fori_loop(0, T, t2body, (zf,) * NVR)

            ro = lax.rem(row, OS)
            for j in range(NVR):
                sl = pl.ds(j * L, L)
                stage_v[ro, sl] = ((accs[j] - corr) * invc_v * gam_v[sl]
                                   + bet_v[sl])

            @pl.when(ro == OS - 1)
            def _():
                base = pl.multiple_of(row0 + row - (OS - 1), OS)
                pltpu.sync_copy(stage_v, out_hbm.at[pl.ds(base, OS)])

        load_chunk(0)
        issue(0, 0, sem0)
        issue(1, 1, sem1)

        def rbody(r2, _):
            ra = 2 * r2
            rb = ra + 1
            wait(0, sem0)
            process(0, ra)

            @pl.when(jnp.logical_and(ra + 2 < rpw, lax.rem(ra + 2, CH) == 0))
            def _():
                load_chunk((ra + 2) // CH)

            @pl.when(ra + 2 < rpw)
            def _():
                issue(0, ra + 2, sem0)

            wait(1, sem1)
            process(1, rb)

            @pl.when(jnp.logical_and(rb + 2 < rpw, lax.rem(rb + 2, CH) == 0))
            def _():
                load_chunk((rb + 2) // CH)

            @pl.when(rb + 2 < rpw)
            def _():
                issue(1, rb + 2, sem1)
            return 0

        lax.fori_loop(0, rpw // 2, rbody, 0)

    mesh = plsc.VectorSubcoreMesh(core_axis_name="c", subcore_axis_name="s")
    return pl.kernel(
        body,
        out_type=jax.ShapeDtypeStruct((R, HID), jnp.float32),
        mesh=mesh,
        scratch_types=[
            pltpu.VMEM((ptrows, HID), jnp.float32),
            pltpu.VMEM((1, HID), jnp.float32),
            pltpu.VMEM((HID,), jnp.float32),
            pltpu.VMEM((HID,), jnp.float32),
            pltpu.VMEM((CH, T), jnp.int32),
            pltpu.VMEM((2, T), jnp.int32),
            pltpu.VMEM((2, T, HID), jnp.float32),
            pltpu.VMEM((1, 48), jnp.int32),
            pltpu.VMEM((OS, HID), jnp.float32),
            pltpu.SemaphoreType.DMA,
            pltpu.SemaphoreType.DMA,
        ],
        name=f"embed_pool_T{T}",
    )


def kernel(x, edge_attr, word_emb, pos_emb, type_emb, ln_gamma, ln_beta):
    ids_x = x[:, 128:144].astype(jnp.int32)
    ids_e = edge_attr[:, 32:64].astype(jnp.int32)
    pooled_x = _make_pooler(x.shape[0], 16)(
        ids_x, word_emb, pos_emb, type_emb, ln_gamma, ln_beta)
    pooled_e = _make_pooler(edge_attr.shape[0], 32)(
        ids_e, word_emb, pos_emb, type_emb, ln_gamma, ln_beta)
    return (jnp.concatenate([x[:, :128], pooled_x], axis=1),
            jnp.concatenate([edge_attr[:, :32], pooled_e], axis=1))
